# baseline (speedup 1.0000x reference)
import functools

import jax
import jax.numpy as jnp
from jax import lax
from jax.experimental import pallas as pl
from jax.experimental.pallas import tpu as pltpu
from jax.experimental.pallas import tpu_sc as plsc

_T, _H, _I, _E = 2048, 1024, 2048, 8
_BN = 128
_K = 2
_BM = 128
_M = _T * _K
_NB = _M // _BM + _E
_NBM = _NB * _BM


def _ffn_body(be_ref, xs_ref, w13_ref, w2_ref, s13_ref, s2_ref, g_ref, o_ref,
              w13q, w2q):
    i = pl.program_id(0)
    changed = jnp.logical_or(
        i == 0, be_ref[i] != be_ref[jnp.maximum(i - 1, 0)])

    @pl.when(changed)
    def _dequant_expert():
        for r in range(2 * _I // _BN):
            w13q[r * _BN:(r + 1) * _BN, :] = (
                w13_ref[0, r * _BN:(r + 1) * _BN, :].astype(jnp.float32)
                * s13_ref[0, r:r + 1, :]
            ).astype(jnp.bfloat16)
        for r in range(_H // _BN):
            w2q[r * _BN:(r + 1) * _BN, :] = (
                w2_ref[0, r * _BN:(r + 1) * _BN, :].astype(jnp.float32)
                * s2_ref[0, r:r + 1, :]
            ).astype(jnp.bfloat16)

    x = xs_ref[...]
    h = lax.dot_general(x, w13q[...], (((1,), (1,)), ((), ())),
                        preferred_element_type=jnp.float32)
    gate = h[:, :_I]
    up = h[:, _I:]
    act = (gate * jax.nn.sigmoid(gate) * up).astype(jnp.bfloat16)
    o = lax.dot_general(act, w2q[...], (((1,), (1,)), ((), ())),
                        preferred_element_type=jnp.float32)
    o_ref[...] = o * g_ref[...]


def _ffn(block_expert, xs, w13, w2, s13k, s2k, gamma):
    grid_spec = pltpu.PrefetchScalarGridSpec(
        num_scalar_prefetch=1,
        grid=(_NB,),
        in_specs=[
            pl.BlockSpec((_BM, _H), lambda i, be: (i, 0)),
            pl.BlockSpec((1, 2 * _I, _H), lambda i, be: (be[i], 0, 0)),
            pl.BlockSpec((1, _H, _I), lambda i, be: (be[i], 0, 0)),
            pl.BlockSpec((1, 2 * _I // _BN, _H), lambda i, be: (be[i], 0, 0)),
            pl.BlockSpec((1, _H // _BN, _I), lambda i, be: (be[i], 0, 0)),
            pl.BlockSpec((_BM, 1), lambda i, be: (i, 0)),
        ],
        out_specs=pl.BlockSpec((_BM, _H), lambda i, be: (i, 0)),
        scratch_shapes=[
            pltpu.VMEM((2 * _I, _H), jnp.bfloat16),
            pltpu.VMEM((_H, _I), jnp.bfloat16),
        ],
    )
    return pl.pallas_call(
        _ffn_body,
        grid_spec=grid_spec,
        out_shape=jax.ShapeDtypeStruct((_NBM, _H), jnp.float32),
    )(block_expert, xs, w13, w2, s13k, s2k, gamma)


_NC, _NS = 2, 16
_NW = _NC * _NS


def _sc_gather_rows(table, idx):
    V, D = table.shape
    B = idx.shape[0]
    b_per_w = B // _NW
    mesh = plsc.VectorSubcoreMesh(core_axis_name="c", subcore_axis_name="s")

    @functools.partial(
        pl.kernel, mesh=mesh,
        out_type=jax.ShapeDtypeStruct((B, D), jnp.float32),
        scratch_types=[
            pltpu.VMEM((b_per_w,), jnp.int32),
            pltpu.VMEM((b_per_w, D), jnp.float32),
            pltpu.SemaphoreType.DMA,
            pltpu.SemaphoreType.DMA,
        ],
    )
    def k(table_hbm, idx_hbm, out_hbm, idx_v, rows_v, sem_a, sem_b):
        wid = lax.axis_index("s") * _NC + lax.axis_index("c")
        base = wid * b_per_w
        half = b_per_w // 2
        pltpu.sync_copy(idx_hbm.at[pl.ds(base, b_per_w)], idx_v)
        cp_a = pltpu.async_copy(
            table_hbm.at[idx_v.at[pl.ds(0, half)]],
            rows_v.at[pl.ds(0, half)], sem_a)
        cp_b = pltpu.async_copy(
            table_hbm.at[idx_v.at[pl.ds(half, half)]],
            rows_v.at[pl.ds(half, half)], sem_b)
        cp_a.wait()
        out_a = pltpu.async_copy(
            rows_v.at[pl.ds(0, half)], out_hbm.at[pl.ds(base, half)], sem_a)
        cp_b.wait()
        pltpu.sync_copy(
            rows_v.at[pl.ds(half, half)], out_hbm.at[pl.ds(base + half, half)])
        out_a.wait()

    return k(table, idx)


def _sc_combine(o_sorted, d0, d1):
    B, D = o_sorted.shape
    n_tok = d0.shape[0]
    t_per_w = n_tok // _NW
    ch = t_per_w // 2
    mesh = plsc.VectorSubcoreMesh(core_axis_name="c", subcore_axis_name="s")

    @functools.partial(
        pl.kernel, mesh=mesh,
        out_type=[jax.ShapeDtypeStruct((n_tok, D), jnp.float32),
                  jax.ShapeDtypeStruct((n_tok, D), jnp.float32)],
        scratch_types=[
            pltpu.VMEM((t_per_w,), jnp.int32),
            pltpu.VMEM((t_per_w,), jnp.int32),
            pltpu.VMEM((ch, D), jnp.float32),
            pltpu.VMEM((ch, D), jnp.float32),
            pltpu.SemaphoreType.DMA,
            pltpu.SemaphoreType.DMA,
        ],
    )
    def k(o_hbm, d0_hbm, d1_hbm, oa_hbm, ob_hbm, i0_v, i1_v, ra_v, rb_v,
          sem_a, sem_b):
        wid = lax.axis_index("s") * _NC + lax.axis_index("c")
        base = wid * t_per_w
        pltpu.sync_copy(d0_hbm.at[pl.ds(base, t_per_w)], i0_v)
        pltpu.sync_copy(d1_hbm.at[pl.ds(base, t_per_w)], i1_v)
        for c in range(t_per_w // ch):
            gb = base + c * ch
            cp_a = pltpu.async_copy(
                o_hbm.at[i0_v.at[pl.ds(c * ch, ch)]], ra_v, sem_a)
            cp_b = pltpu.async_copy(
                o_hbm.at[i1_v.at[pl.ds(c * ch, ch)]], rb_v, sem_b)
            cp_a.wait()
            pltpu.sync_copy(ra_v, oa_hbm.at[pl.ds(gb, ch)])
            cp_b.wait()
            pltpu.sync_copy(rb_v, ob_hbm.at[pl.ds(gb, ch)])

    return k(o_sorted, d0, d1)


def _add_body(a_ref, b_ref, o_ref):
    o_ref[...] = a_ref[...] + b_ref[...]


def _tc_add(a, b):
    n, d = a.shape
    blk = 256
    return pl.pallas_call(
        _add_body,
        grid=(n // blk,),
        in_specs=[pl.BlockSpec((blk, d), lambda i: (i, 0)),
                  pl.BlockSpec((blk, d), lambda i: (i, 0))],
        out_specs=pl.BlockSpec((blk, d), lambda i: (i, 0)),
        out_shape=jax.ShapeDtypeStruct((n, d), jnp.float32),
    )(a, b)


def kernel(x, router_logits, w13_weight, w2_weight, w13_weight_scale_inv,
           w2_weight_scale_inv, top_k, renormalize):
    probs = jax.nn.softmax(router_logits.astype(jnp.float32), axis=-1)
    tw, ti = lax.top_k(probs, _K)
    tw = tw * (jnp.asarray(top_k, jnp.float32) / _K)
    tw = jnp.where(jnp.asarray(renormalize) != 0,
                   tw / jnp.sum(tw, axis=-1, keepdims=True), tw)

    flat_ids = ti.reshape(-1).astype(jnp.int32)
    oh = flat_ids[:, None] == jnp.arange(_E, dtype=jnp.int32)[None, :]
    ohi = oh.astype(jnp.int32)
    counts = ohi.sum(axis=0)
    rank = jnp.where(oh, jnp.cumsum(ohi, axis=0) - 1, 0).sum(axis=1)
    nblk = (counts + _BM - 1) // _BM
    bstart = jnp.concatenate(
        [jnp.zeros((1,), jnp.int32), jnp.cumsum(nblk)[:-1].astype(jnp.int32)])
    dest = bstart[flat_ids] * _BM + rank
    token = jnp.arange(_M, dtype=jnp.int32) // _K
    sorted_token = (jnp.arange(_NBM, dtype=jnp.int32) % _T).at[dest].set(token)
    gamma = jnp.zeros((_NBM, 1), jnp.float32).at[dest, 0].set(tw.reshape(-1))
    bend = jnp.cumsum(nblk)
    block_expert = jnp.sum(
        (jnp.arange(_NB, dtype=jnp.int32)[:, None] >= bend[None, :])
        .astype(jnp.int32), axis=1)
    block_expert = jnp.minimum(block_expert, _E - 1)

    s13k = jnp.repeat(w13_weight_scale_inv, _BN, axis=2)
    s2k = jnp.repeat(w2_weight_scale_inv, _BN, axis=2)

    xq = lax.bitcast_convert_type(
        x.astype(jnp.bfloat16).reshape(_T, _H // 2, 2), jnp.float32)
    xs32 = _sc_gather_rows(xq, sorted_token)
    xs = lax.bitcast_convert_type(xs32, jnp.bfloat16).reshape(_NBM, _H)

    o_sorted = _ffn(block_expert, xs, w13_weight.astype(jnp.bfloat16),
                    w2_weight.astype(jnp.bfloat16), s13k, s2k, gamma)
    d = dest.reshape(_T, _K)
    oa, ob = _sc_combine(o_sorted, d[:, 0], d[:, 1])
    return _tc_add(oa, ob)


# speedup vs baseline: 1.0312x; 1.0312x over previous
import functools

import jax
import jax.numpy as jnp
from jax import lax
from jax.experimental import pallas as pl
from jax.experimental.pallas import tpu as pltpu
from jax.experimental.pallas import tpu_sc as plsc

_T, _H, _I, _E = 2048, 1024, 2048, 8
_BN = 128
_K = 2
_BM = 128
_M = _T * _K
_NB = _M // _BM + _E
_NBM = _NB * _BM


def _dequant_body(w_ref, s_ref, o_ref):
    rows = w_ref.shape[1]
    for r in range(rows // _BN):
        o_ref[0, r * _BN:(r + 1) * _BN, :] = (
            w_ref[0, r * _BN:(r + 1) * _BN, :] * s_ref[0, r:r + 1, :]
        ).astype(jnp.bfloat16)


def _dequant13(w13, s13k):
    return pl.pallas_call(
        _dequant_body,
        grid=(_E, 2),
        in_specs=[
            pl.BlockSpec((1, _I, _H), lambda e, c: (e, c, 0)),
            pl.BlockSpec((1, _I // _BN, _H), lambda e, c: (e, c, 0)),
        ],
        out_specs=pl.BlockSpec((1, _I, _H), lambda e, c: (e, c, 0)),
        out_shape=jax.ShapeDtypeStruct((_E, 2 * _I, _H), jnp.bfloat16),
    )(w13, s13k)


def _dequant2(w2, s2k):
    return pl.pallas_call(
        _dequant_body,
        grid=(_E,),
        in_specs=[
            pl.BlockSpec((1, _H, _I), lambda e: (e, 0, 0)),
            pl.BlockSpec((1, _H // _BN, _I), lambda e: (e, 0, 0)),
        ],
        out_specs=pl.BlockSpec((1, _H, _I), lambda e: (e, 0, 0)),
        out_shape=jax.ShapeDtypeStruct((_E, _H, _I), jnp.bfloat16),
    )(w2, s2k)


def _ffn_body(be_ref, xs_ref, w13_ref, w2_ref, g_ref, o_ref):
    x = xs_ref[...]
    h = lax.dot_general(x, w13_ref[0], (((1,), (1,)), ((), ())),
                        preferred_element_type=jnp.float32)
    gate = h[:, :_I]
    up = h[:, _I:]
    act = (gate * jax.nn.sigmoid(gate) * up).astype(jnp.bfloat16)
    o = lax.dot_general(act, w2_ref[0], (((1,), (1,)), ((), ())),
                        preferred_element_type=jnp.float32)
    o_ref[...] = o * g_ref[...]


def _ffn(block_expert, xs, w13f, w2f, gamma):
    grid_spec = pltpu.PrefetchScalarGridSpec(
        num_scalar_prefetch=1,
        grid=(_NB,),
        in_specs=[
            pl.BlockSpec((_BM, _H), lambda i, be: (i, 0)),
            pl.BlockSpec((1, 2 * _I, _H), lambda i, be: (be[i], 0, 0)),
            pl.BlockSpec((1, _H, _I), lambda i, be: (be[i], 0, 0)),
            pl.BlockSpec((_BM, 1), lambda i, be: (i, 0)),
        ],
        out_specs=pl.BlockSpec((_BM, _H), lambda i, be: (i, 0)),
    )
    return pl.pallas_call(
        _ffn_body,
        grid_spec=grid_spec,
        out_shape=jax.ShapeDtypeStruct((_NBM, _H), jnp.float32),
    )(block_expert, xs, w13f, w2f, gamma)


_NC, _NS = 2, 16
_NW = _NC * _NS


def _sc_gather_rows(table, idx):
    V, D = table.shape
    B = idx.shape[0]
    b_per_w = B // _NW
    mesh = plsc.VectorSubcoreMesh(core_axis_name="c", subcore_axis_name="s")

    @functools.partial(
        pl.kernel, mesh=mesh,
        out_type=jax.ShapeDtypeStruct((B, D), jnp.float32),
        scratch_types=[
            pltpu.VMEM((b_per_w,), jnp.int32),
            pltpu.VMEM((b_per_w, D), jnp.float32),
            pltpu.SemaphoreType.DMA,
            pltpu.SemaphoreType.DMA,
        ],
    )
    def k(table_hbm, idx_hbm, out_hbm, idx_v, rows_v, sem_a, sem_b):
        wid = lax.axis_index("s") * _NC + lax.axis_index("c")
        base = wid * b_per_w
        half = b_per_w // 2
        pltpu.sync_copy(idx_hbm.at[pl.ds(base, b_per_w)], idx_v)
        cp_a = pltpu.async_copy(
            table_hbm.at[idx_v.at[pl.ds(0, half)]],
            rows_v.at[pl.ds(0, half)], sem_a)
        cp_b = pltpu.async_copy(
            table_hbm.at[idx_v.at[pl.ds(half, half)]],
            rows_v.at[pl.ds(half, half)], sem_b)
        cp_a.wait()
        out_a = pltpu.async_copy(
            rows_v.at[pl.ds(0, half)], out_hbm.at[pl.ds(base, half)], sem_a)
        cp_b.wait()
        pltpu.sync_copy(
            rows_v.at[pl.ds(half, half)], out_hbm.at[pl.ds(base + half, half)])
        out_a.wait()

    return k(table, idx)


def _sc_combine(o_sorted, d0, d1):
    B, D = o_sorted.shape
    n_tok = d0.shape[0]
    t_per_w = n_tok // _NW
    ch = t_per_w // 2
    mesh = plsc.VectorSubcoreMesh(core_axis_name="c", subcore_axis_name="s")

    @functools.partial(
        pl.kernel, mesh=mesh,
        out_type=[jax.ShapeDtypeStruct((n_tok, D), jnp.float32),
                  jax.ShapeDtypeStruct((n_tok, D), jnp.float32)],
        scratch_types=[
            pltpu.VMEM((t_per_w,), jnp.int32),
            pltpu.VMEM((t_per_w,), jnp.int32),
            pltpu.VMEM((ch, D), jnp.float32),
            pltpu.VMEM((ch, D), jnp.float32),
            pltpu.SemaphoreType.DMA,
            pltpu.SemaphoreType.DMA,
        ],
    )
    def k(o_hbm, d0_hbm, d1_hbm, oa_hbm, ob_hbm, i0_v, i1_v, ra_v, rb_v,
          sem_a, sem_b):
        wid = lax.axis_index("s") * _NC + lax.axis_index("c")
        base = wid * t_per_w
        pltpu.sync_copy(d0_hbm.at[pl.ds(base, t_per_w)], i0_v)
        pltpu.sync_copy(d1_hbm.at[pl.ds(base, t_per_w)], i1_v)
        for c in range(t_per_w // ch):
            gb = base + c * ch
            cp_a = pltpu.async_copy(
                o_hbm.at[i0_v.at[pl.ds(c * ch, ch)]], ra_v, sem_a)
            cp_b = pltpu.async_copy(
                o_hbm.at[i1_v.at[pl.ds(c * ch, ch)]], rb_v, sem_b)
            cp_a.wait()
            pltpu.sync_copy(ra_v, oa_hbm.at[pl.ds(gb, ch)])
            cp_b.wait()
            pltpu.sync_copy(rb_v, ob_hbm.at[pl.ds(gb, ch)])

    return k(o_sorted, d0, d1)


def _add_body(a_ref, b_ref, o_ref):
    o_ref[...] = a_ref[...] + b_ref[...]


def _tc_add(a, b):
    n, d = a.shape
    blk = 256
    return pl.pallas_call(
        _add_body,
        grid=(n // blk,),
        in_specs=[pl.BlockSpec((blk, d), lambda i: (i, 0)),
                  pl.BlockSpec((blk, d), lambda i: (i, 0))],
        out_specs=pl.BlockSpec((blk, d), lambda i: (i, 0)),
        out_shape=jax.ShapeDtypeStruct((n, d), jnp.float32),
    )(a, b)


def kernel(x, router_logits, w13_weight, w2_weight, w13_weight_scale_inv,
           w2_weight_scale_inv, top_k, renormalize):
    probs = jax.nn.softmax(router_logits.astype(jnp.float32), axis=-1)
    tw, ti = lax.top_k(probs, _K)
    tw = tw * (jnp.asarray(top_k, jnp.float32) / _K)
    tw = jnp.where(jnp.asarray(renormalize) != 0,
                   tw / jnp.sum(tw, axis=-1, keepdims=True), tw)

    flat_ids = ti.reshape(-1).astype(jnp.int32)
    oh = flat_ids[:, None] == jnp.arange(_E, dtype=jnp.int32)[None, :]
    ohi = oh.astype(jnp.int32)
    counts = ohi.sum(axis=0)
    rank = jnp.where(oh, jnp.cumsum(ohi, axis=0) - 1, 0).sum(axis=1)
    nblk = (counts + _BM - 1) // _BM
    bstart = jnp.concatenate(
        [jnp.zeros((1,), jnp.int32), jnp.cumsum(nblk)[:-1].astype(jnp.int32)])
    dest = bstart[flat_ids] * _BM + rank
    token = jnp.arange(_M, dtype=jnp.int32) // _K
    sorted_token = (jnp.arange(_NBM, dtype=jnp.int32) % _T).at[dest].set(token)
    gamma = jnp.zeros((_NBM, 1), jnp.float32).at[dest, 0].set(tw.reshape(-1))
    bend = jnp.cumsum(nblk)
    block_expert = jnp.sum(
        (jnp.arange(_NB, dtype=jnp.int32)[:, None] >= bend[None, :])
        .astype(jnp.int32), axis=1)
    block_expert = jnp.minimum(block_expert, _E - 1)

    s13k = jnp.repeat(w13_weight_scale_inv, _BN, axis=2)
    s2k = jnp.repeat(w2_weight_scale_inv, _BN, axis=2)
    w13f = _dequant13(w13_weight, s13k)
    w2f = _dequant2(w2_weight, s2k)

    xq = lax.bitcast_convert_type(
        x.astype(jnp.bfloat16).reshape(_T, _H // 2, 2), jnp.float32)
    xs32 = _sc_gather_rows(xq, sorted_token)
    xs = lax.bitcast_convert_type(xs32, jnp.bfloat16).reshape(_NBM, _H)

    o_sorted = _ffn(block_expert, xs, w13f, w2f, gamma)
    d = dest.reshape(_T, _K)
    oa, ob = _sc_combine(o_sorted, d[:, 0], d[:, 1])
    return _tc_add(oa, ob)


# speedup vs baseline: 1.1771x; 1.1414x over previous
import functools

import jax
import jax.numpy as jnp
from jax import lax
from jax.experimental import pallas as pl
from jax.experimental.pallas import tpu as pltpu
from jax.experimental.pallas import tpu_sc as plsc

_T, _H, _I, _E = 2048, 1024, 2048, 8
_BN = 128
_K = 2
_BM = 256
_M = _T * _K
_NB = _M // _BM + _E
_NBM = _NB * _BM


def _dequant_body(w_ref, s_ref, o_ref):
    rows = w_ref.shape[1]
    for r in range(rows // _BN):
        o_ref[0, r * _BN:(r + 1) * _BN, :] = (
            w_ref[0, r * _BN:(r + 1) * _BN, :] * s_ref[0, r:r + 1, :]
        ).astype(jnp.bfloat16)


def _dequant13(w13, s13k):
    return pl.pallas_call(
        _dequant_body,
        grid=(_E, 2),
        in_specs=[
            pl.BlockSpec((1, _I, _H), lambda e, c: (e, c, 0)),
            pl.BlockSpec((1, _I // _BN, _H), lambda e, c: (e, c, 0)),
        ],
        out_specs=pl.BlockSpec((1, _I, _H), lambda e, c: (e, c, 0)),
        out_shape=jax.ShapeDtypeStruct((_E, 2 * _I, _H), jnp.bfloat16),
    )(w13, s13k)


def _dequant2(w2, s2k):
    return pl.pallas_call(
        _dequant_body,
        grid=(_E,),
        in_specs=[
            pl.BlockSpec((1, _H, _I), lambda e: (e, 0, 0)),
            pl.BlockSpec((1, _H // _BN, _I), lambda e: (e, 0, 0)),
        ],
        out_specs=pl.BlockSpec((1, _H, _I), lambda e: (e, 0, 0)),
        out_shape=jax.ShapeDtypeStruct((_E, _H, _I), jnp.bfloat16),
    )(w2, s2k)


def _ffn_body(be_ref, xs_ref, w13_ref, w2_ref, g_ref, o_ref):
    x = xs_ref[...]
    h = lax.dot_general(x, w13_ref[0], (((1,), (1,)), ((), ())),
                        preferred_element_type=jnp.float32)
    gate = h[:, :_I]
    up = h[:, _I:]
    act = (gate * jax.nn.sigmoid(gate) * up).astype(jnp.bfloat16)
    o = lax.dot_general(act, w2_ref[0], (((1,), (1,)), ((), ())),
                        preferred_element_type=jnp.float32)
    o_ref[...] = o * g_ref[...]


def _ffn(block_expert, xs, w13f, w2f, gamma):
    grid_spec = pltpu.PrefetchScalarGridSpec(
        num_scalar_prefetch=1,
        grid=(_NB,),
        in_specs=[
            pl.BlockSpec((_BM, _H), lambda i, be: (i, 0)),
            pl.BlockSpec((1, 2 * _I, _H), lambda i, be: (be[i], 0, 0)),
            pl.BlockSpec((1, _H, _I), lambda i, be: (be[i], 0, 0)),
            pl.BlockSpec((_BM, 1), lambda i, be: (i, 0)),
        ],
        out_specs=pl.BlockSpec((_BM, _H), lambda i, be: (i, 0)),
    )
    return pl.pallas_call(
        _ffn_body,
        grid_spec=grid_spec,
        out_shape=jax.ShapeDtypeStruct((_NBM, _H), jnp.float32),
    )(block_expert, xs, w13f, w2f, gamma)


_NC, _NS = 2, 16
_NW = _NC * _NS


def _sc_gather_rows(table, idx):
    V, D = table.shape
    B = idx.shape[0]
    b_per_w = B // _NW
    mesh = plsc.VectorSubcoreMesh(core_axis_name="c", subcore_axis_name="s")

    @functools.partial(
        pl.kernel, mesh=mesh,
        out_type=jax.ShapeDtypeStruct((B, D), jnp.float32),
        scratch_types=[
            pltpu.VMEM((b_per_w,), jnp.int32),
            pltpu.VMEM((b_per_w, D), jnp.float32),
            pltpu.SemaphoreType.DMA,
            pltpu.SemaphoreType.DMA,
        ],
    )
    def k(table_hbm, idx_hbm, out_hbm, idx_v, rows_v, sem_a, sem_b):
        wid = lax.axis_index("s") * _NC + lax.axis_index("c")
        base = wid * b_per_w
        half = b_per_w // 2
        pltpu.sync_copy(idx_hbm.at[pl.ds(base, b_per_w)], idx_v)
        cp_a = pltpu.async_copy(
            table_hbm.at[idx_v.at[pl.ds(0, half)]],
            rows_v.at[pl.ds(0, half)], sem_a)
        cp_b = pltpu.async_copy(
            table_hbm.at[idx_v.at[pl.ds(half, half)]],
            rows_v.at[pl.ds(half, half)], sem_b)
        cp_a.wait()
        out_a = pltpu.async_copy(
            rows_v.at[pl.ds(0, half)], out_hbm.at[pl.ds(base, half)], sem_a)
        cp_b.wait()
        pltpu.sync_copy(
            rows_v.at[pl.ds(half, half)], out_hbm.at[pl.ds(base + half, half)])
        out_a.wait()

    return k(table, idx)


def _sc_combine(o_sorted, d0, d1):
    B, D = o_sorted.shape
    n_tok = d0.shape[0]
    t_per_w = n_tok // _NW
    ch = t_per_w // 2
    mesh = plsc.VectorSubcoreMesh(core_axis_name="c", subcore_axis_name="s")

    @functools.partial(
        pl.kernel, mesh=mesh,
        out_type=[jax.ShapeDtypeStruct((n_tok, D), jnp.float32),
                  jax.ShapeDtypeStruct((n_tok, D), jnp.float32)],
        scratch_types=[
            pltpu.VMEM((t_per_w,), jnp.int32),
            pltpu.VMEM((t_per_w,), jnp.int32),
            pltpu.VMEM((ch, D), jnp.float32),
            pltpu.VMEM((ch, D), jnp.float32),
            pltpu.SemaphoreType.DMA,
            pltpu.SemaphoreType.DMA,
        ],
    )
    def k(o_hbm, d0_hbm, d1_hbm, oa_hbm, ob_hbm, i0_v, i1_v, ra_v, rb_v,
          sem_a, sem_b):
        wid = lax.axis_index("s") * _NC + lax.axis_index("c")
        base = wid * t_per_w
        pltpu.sync_copy(d0_hbm.at[pl.ds(base, t_per_w)], i0_v)
        pltpu.sync_copy(d1_hbm.at[pl.ds(base, t_per_w)], i1_v)
        for c in range(t_per_w // ch):
            gb = base + c * ch
            cp_a = pltpu.async_copy(
                o_hbm.at[i0_v.at[pl.ds(c * ch, ch)]], ra_v, sem_a)
            cp_b = pltpu.async_copy(
                o_hbm.at[i1_v.at[pl.ds(c * ch, ch)]], rb_v, sem_b)
            cp_a.wait()
            pltpu.sync_copy(ra_v, oa_hbm.at[pl.ds(gb, ch)])
            cp_b.wait()
            pltpu.sync_copy(rb_v, ob_hbm.at[pl.ds(gb, ch)])

    return k(o_sorted, d0, d1)


def _add_body(a_ref, b_ref, o_ref):
    o_ref[...] = a_ref[...] + b_ref[...]


def _tc_add(a, b):
    n, d = a.shape
    blk = 256
    return pl.pallas_call(
        _add_body,
        grid=(n // blk,),
        in_specs=[pl.BlockSpec((blk, d), lambda i: (i, 0)),
                  pl.BlockSpec((blk, d), lambda i: (i, 0))],
        out_specs=pl.BlockSpec((blk, d), lambda i: (i, 0)),
        out_shape=jax.ShapeDtypeStruct((n, d), jnp.float32),
    )(a, b)


def kernel(x, router_logits, w13_weight, w2_weight, w13_weight_scale_inv,
           w2_weight_scale_inv, top_k, renormalize):
    probs = jax.nn.softmax(router_logits.astype(jnp.float32), axis=-1)
    tw, ti = lax.top_k(probs, _K)
    tw = tw * (jnp.asarray(top_k, jnp.float32) / _K)
    tw = jnp.where(jnp.asarray(renormalize) != 0,
                   tw / jnp.sum(tw, axis=-1, keepdims=True), tw)

    flat_ids = ti.reshape(-1).astype(jnp.int32)
    oh = flat_ids[:, None] == jnp.arange(_E, dtype=jnp.int32)[None, :]
    ohi = oh.astype(jnp.int32)
    counts = ohi.sum(axis=0)
    rank = jnp.where(oh, jnp.cumsum(ohi, axis=0) - 1, 0).sum(axis=1)
    nblk = (counts + _BM - 1) // _BM
    bstart = jnp.concatenate(
        [jnp.zeros((1,), jnp.int32), jnp.cumsum(nblk)[:-1].astype(jnp.int32)])
    dest = bstart[flat_ids] * _BM + rank
    token = jnp.arange(_M, dtype=jnp.int32) // _K
    sorted_token = (jnp.arange(_NBM, dtype=jnp.int32) % _T).at[dest].set(token)
    gamma = jnp.zeros((_NBM, 1), jnp.float32).at[dest, 0].set(tw.reshape(-1))
    bend = jnp.cumsum(nblk)
    block_expert = jnp.sum(
        (jnp.arange(_NB, dtype=jnp.int32)[:, None] >= bend[None, :])
        .astype(jnp.int32), axis=1)
    block_expert = jnp.minimum(block_expert, _E - 1)

    s13k = jnp.repeat(w13_weight_scale_inv, _BN, axis=2)
    s2k = jnp.repeat(w2_weight_scale_inv, _BN, axis=2)
    w13f = _dequant13(w13_weight, s13k)
    w2f = _dequant2(w2_weight, s2k)

    xq = lax.bitcast_convert_type(
        x.astype(jnp.bfloat16).reshape(_T, _H // 2, 2), jnp.float32)
    xs32 = _sc_gather_rows(xq, sorted_token)
    xs = lax.bitcast_convert_type(xs32, jnp.bfloat16).reshape(_NBM, _H)

    o_sorted = _ffn(block_expert, xs, w13f, w2f, gamma)
    d = dest.reshape(_T, _K)
    oa, ob = _sc_combine(o_sorted, d[:, 0], d[:, 1])
    return _tc_add(oa, ob)


# speedup vs baseline: 1.8342x; 1.5583x over previous
import functools

import jax
import jax.numpy as jnp
from jax import lax
from jax.experimental import pallas as pl
from jax.experimental.pallas import tpu as pltpu
from jax.experimental.pallas import tpu_sc as plsc

_T, _H, _I, _E = 2048, 1024, 2048, 8
_BN = 128
_K = 2
_BM = 256
_M = _T * _K
_NB = _M // _BM + _E
_NBM = _NB * _BM


def _dequant_body(w_ref, s_ref, o_ref):
    rows = w_ref.shape[1]
    for r in range(rows // _BN):
        o_ref[0, r * _BN:(r + 1) * _BN, :] = (
            w_ref[0, r * _BN:(r + 1) * _BN, :] * s_ref[0, r:r + 1, :]
        ).astype(jnp.bfloat16)


def _dequant13(w13, s13k):
    return pl.pallas_call(
        _dequant_body,
        grid=(_E, 2),
        in_specs=[
            pl.BlockSpec((1, _I, _H), lambda e, c: (e, c, 0)),
            pl.BlockSpec((1, _I // _BN, _H), lambda e, c: (e, c, 0)),
        ],
        out_specs=pl.BlockSpec((1, _I, _H), lambda e, c: (e, c, 0)),
        out_shape=jax.ShapeDtypeStruct((_E, 2 * _I, _H), jnp.bfloat16),
    )(w13, s13k)


def _dequant2(w2, s2k):
    return pl.pallas_call(
        _dequant_body,
        grid=(_E,),
        in_specs=[
            pl.BlockSpec((1, _H, _I), lambda e: (e, 0, 0)),
            pl.BlockSpec((1, _H // _BN, _I), lambda e: (e, 0, 0)),
        ],
        out_specs=pl.BlockSpec((1, _H, _I), lambda e: (e, 0, 0)),
        out_shape=jax.ShapeDtypeStruct((_E, _H, _I), jnp.bfloat16),
    )(w2, s2k)


def _ffn_body(be_ref, xs_ref, w13_ref, w2_ref, g_ref, o_ref):
    x = xs_ref[...].astype(jnp.bfloat16)
    h = lax.dot_general(x, w13_ref[0], (((1,), (1,)), ((), ())),
                        preferred_element_type=jnp.float32)
    gate = h[:, :_I]
    up = h[:, _I:]
    act = (gate * jax.nn.sigmoid(gate) * up).astype(jnp.bfloat16)
    o = lax.dot_general(act, w2_ref[0], (((1,), (1,)), ((), ())),
                        preferred_element_type=jnp.float32)
    o_ref[...] = o * g_ref[...]


def _ffn(block_expert, xs, w13f, w2f, gamma):
    grid_spec = pltpu.PrefetchScalarGridSpec(
        num_scalar_prefetch=1,
        grid=(_NB,),
        in_specs=[
            pl.BlockSpec((_BM, _H), lambda i, be: (i, 0)),
            pl.BlockSpec((1, 2 * _I, _H), lambda i, be: (be[i], 0, 0)),
            pl.BlockSpec((1, _H, _I), lambda i, be: (be[i], 0, 0)),
            pl.BlockSpec((_BM, 1), lambda i, be: (i, 0)),
        ],
        out_specs=pl.BlockSpec((_BM, _H), lambda i, be: (i, 0)),
    )
    return pl.pallas_call(
        _ffn_body,
        grid_spec=grid_spec,
        out_shape=jax.ShapeDtypeStruct((_NBM, _H), jnp.float32),
    )(block_expert, xs, w13f, w2f, gamma)


_NC, _NS = 2, 16
_NW = _NC * _NS


def _sc_gather_rows(table, idx, nch=4):
    V, D = table.shape
    B = idx.shape[0]
    b_per_w = B // _NW
    ch = b_per_w // nch
    mesh = plsc.VectorSubcoreMesh(core_axis_name="c", subcore_axis_name="s")

    @functools.partial(
        pl.kernel, mesh=mesh,
        out_type=jax.ShapeDtypeStruct((B, D), jnp.float32),
        scratch_types=[
            pltpu.VMEM((b_per_w,), jnp.int32),
            pltpu.VMEM((ch, D), jnp.float32),
            pltpu.VMEM((ch, D), jnp.float32),
            pltpu.SemaphoreType.DMA,
            pltpu.SemaphoreType.DMA,
        ],
    )
    def k(table_hbm, idx_hbm, out_hbm, idx_v, ra_v, rb_v, sem_a, sem_b):
        wid = lax.axis_index("s") * _NC + lax.axis_index("c")
        base = wid * b_per_w
        pltpu.sync_copy(idx_hbm.at[pl.ds(base, b_per_w)], idx_v)
        bufs = [(ra_v, sem_a), (rb_v, sem_b)]
        cps = [None, None]
        cps[0] = pltpu.async_copy(
            table_hbm.at[idx_v.at[pl.ds(0, ch)]], ra_v, sem_a)
        for c in range(nch):
            buf, sem = bufs[c % 2]
            nbuf, nsem = bufs[(c + 1) % 2]
            if c + 1 < nch:
                cps[(c + 1) % 2] = pltpu.async_copy(
                    table_hbm.at[idx_v.at[pl.ds((c + 1) * ch, ch)]],
                    nbuf, nsem)
            cps[c % 2].wait()
            pltpu.sync_copy(buf, out_hbm.at[pl.ds(base + c * ch, ch)])

    return k(table, idx)


def _sc_combine(o_sorted, d0, d1):
    B, D = o_sorted.shape
    n_tok = d0.shape[0]
    t_per_w = n_tok // _NW
    ch = t_per_w // 2
    mesh = plsc.VectorSubcoreMesh(core_axis_name="c", subcore_axis_name="s")

    @functools.partial(
        pl.kernel, mesh=mesh,
        out_type=[jax.ShapeDtypeStruct((n_tok, D), jnp.float32),
                  jax.ShapeDtypeStruct((n_tok, D), jnp.float32)],
        scratch_types=[
            pltpu.VMEM((t_per_w,), jnp.int32),
            pltpu.VMEM((t_per_w,), jnp.int32),
            pltpu.VMEM((ch, D), jnp.float32),
            pltpu.VMEM((ch, D), jnp.float32),
            pltpu.SemaphoreType.DMA,
            pltpu.SemaphoreType.DMA,
        ],
    )
    def k(o_hbm, d0_hbm, d1_hbm, oa_hbm, ob_hbm, i0_v, i1_v, ra_v, rb_v,
          sem_a, sem_b):
        wid = lax.axis_index("s") * _NC + lax.axis_index("c")
        base = wid * t_per_w
        pltpu.sync_copy(d0_hbm.at[pl.ds(base, t_per_w)], i0_v)
        pltpu.sync_copy(d1_hbm.at[pl.ds(base, t_per_w)], i1_v)
        for c in range(t_per_w // ch):
            gb = base + c * ch
            cp_a = pltpu.async_copy(
                o_hbm.at[i0_v.at[pl.ds(c * ch, ch)]], ra_v, sem_a)
            cp_b = pltpu.async_copy(
                o_hbm.at[i1_v.at[pl.ds(c * ch, ch)]], rb_v, sem_b)
            cp_a.wait()
            pltpu.sync_copy(ra_v, oa_hbm.at[pl.ds(gb, ch)])
            cp_b.wait()
            pltpu.sync_copy(rb_v, ob_hbm.at[pl.ds(gb, ch)])

    return k(o_sorted, d0, d1)


def _add_body(a_ref, b_ref, o_ref):
    o_ref[...] = a_ref[...] + b_ref[...]


def _tc_add(a, b):
    n, d = a.shape
    blk = 256
    return pl.pallas_call(
        _add_body,
        grid=(n // blk,),
        in_specs=[pl.BlockSpec((blk, d), lambda i: (i, 0)),
                  pl.BlockSpec((blk, d), lambda i: (i, 0))],
        out_specs=pl.BlockSpec((blk, d), lambda i: (i, 0)),
        out_shape=jax.ShapeDtypeStruct((n, d), jnp.float32),
    )(a, b)


def kernel(x, router_logits, w13_weight, w2_weight, w13_weight_scale_inv,
           w2_weight_scale_inv, top_k, renormalize):
    probs = jax.nn.softmax(router_logits.astype(jnp.float32), axis=-1)
    tw, ti = lax.top_k(probs, _K)
    tw = tw * (jnp.asarray(top_k, jnp.float32) / _K)
    tw = jnp.where(jnp.asarray(renormalize) != 0,
                   tw / jnp.sum(tw, axis=-1, keepdims=True), tw)

    flat_ids = ti.reshape(-1).astype(jnp.int32)
    oh = flat_ids[:, None] == jnp.arange(_E, dtype=jnp.int32)[None, :]
    ohi = oh.astype(jnp.int32)
    counts = ohi.sum(axis=0)
    rank = jnp.where(oh, jnp.cumsum(ohi, axis=0) - 1, 0).sum(axis=1)
    nblk = (counts + _BM - 1) // _BM
    bstart = jnp.concatenate(
        [jnp.zeros((1,), jnp.int32), jnp.cumsum(nblk)[:-1].astype(jnp.int32)])
    dest = bstart[flat_ids] * _BM + rank
    token = jnp.arange(_M, dtype=jnp.int32) // _K
    sorted_token = (jnp.arange(_NBM, dtype=jnp.int32) % _T).at[dest].set(token)
    gamma = jnp.zeros((_NBM, 1), jnp.float32).at[dest, 0].set(tw.reshape(-1))
    bend = jnp.cumsum(nblk)
    block_expert = jnp.sum(
        (jnp.arange(_NB, dtype=jnp.int32)[:, None] >= bend[None, :])
        .astype(jnp.int32), axis=1)
    block_expert = jnp.minimum(block_expert, _E - 1)

    s13k = jnp.repeat(w13_weight_scale_inv, _BN, axis=2)
    s2k = jnp.repeat(w2_weight_scale_inv, _BN, axis=2)
    w13f = _dequant13(w13_weight, s13k)
    w2f = _dequant2(w2_weight, s2k)

    xs = _sc_gather_rows(x, sorted_token)

    o_sorted = _ffn(block_expert, xs, w13f, w2f, gamma)
    d = dest.reshape(_T, _K)
    oa, ob = _sc_combine(o_sorted, d[:, 0], d[:, 1])
    return _tc_add(oa, ob)


# speedup vs baseline: 1.8506x; 1.0089x over previous
import functools

import jax
import jax.numpy as jnp
from jax import lax
from jax.experimental import pallas as pl
from jax.experimental.pallas import tpu as pltpu
from jax.experimental.pallas import tpu_sc as plsc

_T, _H, _I, _E = 2048, 1024, 2048, 8
_BN = 128
_K = 2
_BM = 256
_M = _T * _K
_NB = _M // _BM + _E
_NBM = _NB * _BM


def _dequant_body(w_ref, s_ref, o_ref):
    rows = w_ref.shape[1]
    for r in range(rows // _BN):
        o_ref[0, r * _BN:(r + 1) * _BN, :] = (
            w_ref[0, r * _BN:(r + 1) * _BN, :] * s_ref[0, r:r + 1, :]
        ).astype(jnp.bfloat16)


def _dequant13(w13, s13k):
    return pl.pallas_call(
        _dequant_body,
        grid=(_E, 2),
        in_specs=[
            pl.BlockSpec((1, _I, _H), lambda e, c: (e, c, 0)),
            pl.BlockSpec((1, _I // _BN, _H), lambda e, c: (e, c, 0)),
        ],
        out_specs=pl.BlockSpec((1, _I, _H), lambda e, c: (e, c, 0)),
        out_shape=jax.ShapeDtypeStruct((_E, 2 * _I, _H), jnp.bfloat16),
    )(w13, s13k)


def _dequant2(w2, s2k):
    return pl.pallas_call(
        _dequant_body,
        grid=(_E,),
        in_specs=[
            pl.BlockSpec((1, _H, _I), lambda e: (e, 0, 0)),
            pl.BlockSpec((1, _H // _BN, _I), lambda e: (e, 0, 0)),
        ],
        out_specs=pl.BlockSpec((1, _H, _I), lambda e: (e, 0, 0)),
        out_shape=jax.ShapeDtypeStruct((_E, _H, _I), jnp.bfloat16),
    )(w2, s2k)


def _ffn_body(be_ref, xs_ref, w13_ref, w2_ref, o_ref):
    x = xs_ref[...].astype(jnp.bfloat16)
    h = lax.dot_general(x, w13_ref[0], (((1,), (1,)), ((), ())),
                        preferred_element_type=jnp.float32)
    gate = h[:, :_I]
    up = h[:, _I:]
    act = (gate * jax.nn.sigmoid(gate) * up).astype(jnp.bfloat16)
    o_ref[...] = lax.dot_general(act, w2_ref[0], (((1,), (1,)), ((), ())),
                                 preferred_element_type=jnp.float32)


def _ffn(block_expert, xs, w13f, w2f):
    grid_spec = pltpu.PrefetchScalarGridSpec(
        num_scalar_prefetch=1,
        grid=(_NB,),
        in_specs=[
            pl.BlockSpec((_BM, _H), lambda i, be: (i, 0)),
            pl.BlockSpec((1, 2 * _I, _H), lambda i, be: (be[i], 0, 0)),
            pl.BlockSpec((1, _H, _I), lambda i, be: (be[i], 0, 0)),
        ],
        out_specs=pl.BlockSpec((_BM, _H), lambda i, be: (i, 0)),
    )
    return pl.pallas_call(
        _ffn_body,
        grid_spec=grid_spec,
        out_shape=jax.ShapeDtypeStruct((_NBM, _H), jnp.float32),
    )(block_expert, xs, w13f, w2f)


_NC, _NS = 2, 16
_NW = _NC * _NS


def _sc_gather_rows(table, idx, nch=4):
    V, D = table.shape
    B = idx.shape[0]
    b_per_w = B // _NW
    ch = b_per_w // nch
    mesh = plsc.VectorSubcoreMesh(core_axis_name="c", subcore_axis_name="s")

    @functools.partial(
        pl.kernel, mesh=mesh,
        out_type=jax.ShapeDtypeStruct((B, D), jnp.float32),
        scratch_types=[
            pltpu.VMEM((b_per_w,), jnp.int32),
            pltpu.VMEM((ch, D), jnp.float32),
            pltpu.VMEM((ch, D), jnp.float32),
            pltpu.SemaphoreType.DMA,
            pltpu.SemaphoreType.DMA,
        ],
    )
    def k(table_hbm, idx_hbm, out_hbm, idx_v, ra_v, rb_v, sem_a, sem_b):
        wid = lax.axis_index("s") * _NC + lax.axis_index("c")
        base = wid * b_per_w
        pltpu.sync_copy(idx_hbm.at[pl.ds(base, b_per_w)], idx_v)
        bufs = [(ra_v, sem_a), (rb_v, sem_b)]
        cps = [None, None]
        cps[0] = pltpu.async_copy(
            table_hbm.at[idx_v.at[pl.ds(0, ch)]], ra_v, sem_a)
        for c in range(nch):
            buf, sem = bufs[c % 2]
            nbuf, nsem = bufs[(c + 1) % 2]
            if c + 1 < nch:
                cps[(c + 1) % 2] = pltpu.async_copy(
                    table_hbm.at[idx_v.at[pl.ds((c + 1) * ch, ch)]],
                    nbuf, nsem)
            cps[c % 2].wait()
            pltpu.sync_copy(buf, out_hbm.at[pl.ds(base + c * ch, ch)])

    return k(table, idx)


def _sc_combine(o_sorted, d0, d1):
    B, D = o_sorted.shape
    n_tok = d0.shape[0]
    t_per_w = n_tok // _NW
    ch = t_per_w // 2
    mesh = plsc.VectorSubcoreMesh(core_axis_name="c", subcore_axis_name="s")

    @functools.partial(
        pl.kernel, mesh=mesh,
        out_type=[jax.ShapeDtypeStruct((n_tok, D), jnp.float32),
                  jax.ShapeDtypeStruct((n_tok, D), jnp.float32)],
        scratch_types=[
            pltpu.VMEM((t_per_w,), jnp.int32),
            pltpu.VMEM((t_per_w,), jnp.int32),
            pltpu.VMEM((ch, D), jnp.float32),
            pltpu.VMEM((ch, D), jnp.float32),
            pltpu.SemaphoreType.DMA,
            pltpu.SemaphoreType.DMA,
        ],
    )
    def k(o_hbm, d0_hbm, d1_hbm, oa_hbm, ob_hbm, i0_v, i1_v, ra_v, rb_v,
          sem_a, sem_b):
        wid = lax.axis_index("s") * _NC + lax.axis_index("c")
        base = wid * t_per_w
        pltpu.sync_copy(d0_hbm.at[pl.ds(base, t_per_w)], i0_v)
        pltpu.sync_copy(d1_hbm.at[pl.ds(base, t_per_w)], i1_v)
        for c in range(t_per_w // ch):
            gb = base + c * ch
            cp_a = pltpu.async_copy(
                o_hbm.at[i0_v.at[pl.ds(c * ch, ch)]], ra_v, sem_a)
            cp_b = pltpu.async_copy(
                o_hbm.at[i1_v.at[pl.ds(c * ch, ch)]], rb_v, sem_b)
            cp_a.wait()
            pltpu.sync_copy(ra_v, oa_hbm.at[pl.ds(gb, ch)])
            cp_b.wait()
            pltpu.sync_copy(rb_v, ob_hbm.at[pl.ds(gb, ch)])

    return k(o_sorted, d0, d1)


def _add_body(a_ref, b_ref, g0_ref, g1_ref, o_ref):
    o_ref[...] = a_ref[...] * g0_ref[...] + b_ref[...] * g1_ref[...]


def _tc_scale_add(a, b, g0, g1):
    n, d = a.shape
    blk = 256
    return pl.pallas_call(
        _add_body,
        grid=(n // blk,),
        in_specs=[pl.BlockSpec((blk, d), lambda i: (i, 0)),
                  pl.BlockSpec((blk, d), lambda i: (i, 0)),
                  pl.BlockSpec((blk, 1), lambda i: (i, 0)),
                  pl.BlockSpec((blk, 1), lambda i: (i, 0))],
        out_specs=pl.BlockSpec((blk, d), lambda i: (i, 0)),
        out_shape=jax.ShapeDtypeStruct((n, d), jnp.float32),
    )(a, b, g0, g1)


def kernel(x, router_logits, w13_weight, w2_weight, w13_weight_scale_inv,
           w2_weight_scale_inv, top_k, renormalize):
    probs = jax.nn.softmax(router_logits.astype(jnp.float32), axis=-1)
    tw, ti = lax.top_k(probs, _K)
    tw = tw * (jnp.asarray(top_k, jnp.float32) / _K)
    tw = jnp.where(jnp.asarray(renormalize) != 0,
                   tw / jnp.sum(tw, axis=-1, keepdims=True), tw)

    flat_ids = ti.reshape(-1).astype(jnp.int32)
    oh = flat_ids[:, None] == jnp.arange(_E, dtype=jnp.int32)[None, :]
    ohi = oh.astype(jnp.int32)
    counts = ohi.sum(axis=0)
    rank = jnp.where(oh, jnp.cumsum(ohi, axis=0) - 1, 0).sum(axis=1)
    nblk = (counts + _BM - 1) // _BM
    bstart = jnp.concatenate(
        [jnp.zeros((1,), jnp.int32), jnp.cumsum(nblk)[:-1].astype(jnp.int32)])
    dest = bstart[flat_ids] * _BM + rank
    token = jnp.arange(_M, dtype=jnp.int32) // _K
    sorted_token = (jnp.arange(_NBM, dtype=jnp.int32) % _T).at[dest].set(token)
    bend = jnp.cumsum(nblk)
    block_expert = jnp.sum(
        (jnp.arange(_NB, dtype=jnp.int32)[:, None] >= bend[None, :])
        .astype(jnp.int32), axis=1)
    block_expert = jnp.minimum(block_expert, _E - 1)

    s13k = jnp.repeat(w13_weight_scale_inv, _BN, axis=2)
    s2k = jnp.repeat(w2_weight_scale_inv, _BN, axis=2)
    w13f = _dequant13(w13_weight, s13k)
    w2f = _dequant2(w2_weight, s2k)

    xs = _sc_gather_rows(x, sorted_token)

    o_sorted = _ffn(block_expert, xs, w13f, w2f)
    d = dest.reshape(_T, _K)
    oa, ob = _sc_combine(o_sorted, d[:, 0], d[:, 1])
    return _tc_scale_add(oa, ob, tw[:, :1], tw[:, 1:])


# speedup vs baseline: 1.8666x; 1.0087x over previous
import functools

import jax
import jax.numpy as jnp
from jax import lax
from jax.experimental import pallas as pl
from jax.experimental.pallas import tpu as pltpu
from jax.experimental.pallas import tpu_sc as plsc

_T, _H, _I, _E = 2048, 1024, 2048, 8
_BN = 128
_K = 2
_BM = 256
_M = _T * _K
_NB = _M // _BM + _E
_NBM = _NB * _BM


def _dequant_body(w_ref, s_ref, o_ref):
    rows = w_ref.shape[1]
    for r in range(rows // _BN):
        o_ref[0, r * _BN:(r + 1) * _BN, :] = (
            w_ref[0, r * _BN:(r + 1) * _BN, :] * s_ref[0, r:r + 1, :]
        ).astype(jnp.bfloat16)


def _dequant_both_body(w13_ref, s13_ref, w2_ref, s2_ref, o13_ref, o2_ref):
    _dequant_body(w13_ref, s13_ref, o13_ref)
    for r in range(_H // 2 // _BN):
        o2_ref[0, r * _BN:(r + 1) * _BN, :] = (
            w2_ref[0, r * _BN:(r + 1) * _BN, :] * s2_ref[0, 0, r:r + 1, :]
        ).astype(jnp.bfloat16)


def _dequant_all(w13, s13k, w2, s2k):
    return pl.pallas_call(
        _dequant_both_body,
        grid=(_E, 2),
        in_specs=[
            pl.BlockSpec((1, _I, _H), lambda e, c: (e, c, 0)),
            pl.BlockSpec((1, _I // _BN, _H), lambda e, c: (e, c, 0)),
            pl.BlockSpec((1, _H // 2, _I), lambda e, c: (e, c, 0)),
            pl.BlockSpec((1, 1, _H // 2 // _BN, _I), lambda e, c: (e, c, 0, 0)),
        ],
        out_specs=[
            pl.BlockSpec((1, _I, _H), lambda e, c: (e, c, 0)),
            pl.BlockSpec((1, _H // 2, _I), lambda e, c: (e, c, 0)),
        ],
        out_shape=[
            jax.ShapeDtypeStruct((_E, 2 * _I, _H), jnp.bfloat16),
            jax.ShapeDtypeStruct((_E, _H, _I), jnp.bfloat16),
        ],
    )(w13, s13k, w2,
      s2k.reshape(_E, 2, _H // 2 // _BN, _I))


def _ffn_body(be_ref, xs_ref, w13_ref, w2_ref, o_ref):
    x = xs_ref[...].astype(jnp.bfloat16)
    h = lax.dot_general(x, w13_ref[0], (((1,), (1,)), ((), ())),
                        preferred_element_type=jnp.float32)
    gate = h[:, :_I]
    up = h[:, _I:]
    act = (gate * jax.nn.sigmoid(gate) * up).astype(jnp.bfloat16)
    o_ref[...] = lax.dot_general(act, w2_ref[0], (((1,), (1,)), ((), ())),
                                 preferred_element_type=jnp.float32)


def _ffn(block_expert, xs, w13f, w2f):
    grid_spec = pltpu.PrefetchScalarGridSpec(
        num_scalar_prefetch=1,
        grid=(_NB,),
        in_specs=[
            pl.BlockSpec((_BM, _H), lambda i, be: (i, 0)),
            pl.BlockSpec((1, 2 * _I, _H), lambda i, be: (be[i], 0, 0)),
            pl.BlockSpec((1, _H, _I), lambda i, be: (be[i], 0, 0)),
        ],
        out_specs=pl.BlockSpec((_BM, _H), lambda i, be: (i, 0)),
    )
    return pl.pallas_call(
        _ffn_body,
        grid_spec=grid_spec,
        out_shape=jax.ShapeDtypeStruct((_NBM, _H), jnp.float32),
    )(block_expert, xs, w13f, w2f)


_NC, _NS = 2, 16
_NW = _NC * _NS


def _sc_gather_rows(table, idx, nch=4):
    V, D = table.shape
    B = idx.shape[0]
    b_per_w = B // _NW
    ch = b_per_w // nch
    mesh = plsc.VectorSubcoreMesh(core_axis_name="c", subcore_axis_name="s")

    @functools.partial(
        pl.kernel, mesh=mesh,
        out_type=jax.ShapeDtypeStruct((B, D), jnp.float32),
        scratch_types=[
            pltpu.VMEM((b_per_w,), jnp.int32),
            pltpu.VMEM((ch, D), jnp.float32),
            pltpu.VMEM((ch, D), jnp.float32),
            pltpu.SemaphoreType.DMA,
            pltpu.SemaphoreType.DMA,
        ],
    )
    def k(table_hbm, idx_hbm, out_hbm, idx_v, ra_v, rb_v, sem_a, sem_b):
        wid = lax.axis_index("s") * _NC + lax.axis_index("c")
        base = wid * b_per_w
        pltpu.sync_copy(idx_hbm.at[pl.ds(base, b_per_w)], idx_v)
        bufs = [(ra_v, sem_a), (rb_v, sem_b)]
        cps = [None, None]
        cps[0] = pltpu.async_copy(
            table_hbm.at[idx_v.at[pl.ds(0, ch)]], ra_v, sem_a)
        for c in range(nch):
            buf, sem = bufs[c % 2]
            nbuf, nsem = bufs[(c + 1) % 2]
            if c + 1 < nch:
                cps[(c + 1) % 2] = pltpu.async_copy(
                    table_hbm.at[idx_v.at[pl.ds((c + 1) * ch, ch)]],
                    nbuf, nsem)
            cps[c % 2].wait()
            pltpu.sync_copy(buf, out_hbm.at[pl.ds(base + c * ch, ch)])

    return k(table, idx)


def _sc_combine(o_sorted, d0, d1):
    B, D = o_sorted.shape
    n_tok = d0.shape[0]
    t_per_w = n_tok // _NW
    ch = t_per_w // 2
    mesh = plsc.VectorSubcoreMesh(core_axis_name="c", subcore_axis_name="s")

    @functools.partial(
        pl.kernel, mesh=mesh,
        out_type=[jax.ShapeDtypeStruct((n_tok, D), jnp.float32),
                  jax.ShapeDtypeStruct((n_tok, D), jnp.float32)],
        scratch_types=[
            pltpu.VMEM((t_per_w,), jnp.int32),
            pltpu.VMEM((t_per_w,), jnp.int32),
            pltpu.VMEM((ch, D), jnp.float32),
            pltpu.VMEM((ch, D), jnp.float32),
            pltpu.SemaphoreType.DMA,
            pltpu.SemaphoreType.DMA,
        ],
    )
    def k(o_hbm, d0_hbm, d1_hbm, oa_hbm, ob_hbm, i0_v, i1_v, ra_v, rb_v,
          sem_a, sem_b):
        wid = lax.axis_index("s") * _NC + lax.axis_index("c")
        base = wid * t_per_w
        pltpu.sync_copy(d0_hbm.at[pl.ds(base, t_per_w)], i0_v)
        pltpu.sync_copy(d1_hbm.at[pl.ds(base, t_per_w)], i1_v)
        for c in range(t_per_w // ch):
            gb = base + c * ch
            cp_a = pltpu.async_copy(
                o_hbm.at[i0_v.at[pl.ds(c * ch, ch)]], ra_v, sem_a)
            cp_b = pltpu.async_copy(
                o_hbm.at[i1_v.at[pl.ds(c * ch, ch)]], rb_v, sem_b)
            cp_a.wait()
            pltpu.sync_copy(ra_v, oa_hbm.at[pl.ds(gb, ch)])
            cp_b.wait()
            pltpu.sync_copy(rb_v, ob_hbm.at[pl.ds(gb, ch)])

    return k(o_sorted, d0, d1)


def _add_body(a_ref, b_ref, g0_ref, g1_ref, o_ref):
    o_ref[...] = a_ref[...] * g0_ref[...] + b_ref[...] * g1_ref[...]


def _tc_scale_add(a, b, g0, g1):
    n, d = a.shape
    blk = 256
    return pl.pallas_call(
        _add_body,
        grid=(n // blk,),
        in_specs=[pl.BlockSpec((blk, d), lambda i: (i, 0)),
                  pl.BlockSpec((blk, d), lambda i: (i, 0)),
                  pl.BlockSpec((blk, 1), lambda i: (i, 0)),
                  pl.BlockSpec((blk, 1), lambda i: (i, 0))],
        out_specs=pl.BlockSpec((blk, d), lambda i: (i, 0)),
        out_shape=jax.ShapeDtypeStruct((n, d), jnp.float32),
    )(a, b, g0, g1)


def kernel(x, router_logits, w13_weight, w2_weight, w13_weight_scale_inv,
           w2_weight_scale_inv, top_k, renormalize):
    probs = jax.nn.softmax(router_logits.astype(jnp.float32), axis=-1)
    tw, ti = lax.top_k(probs, _K)
    tw = tw * (jnp.asarray(top_k, jnp.float32) / _K)
    tw = jnp.where(jnp.asarray(renormalize) != 0,
                   tw / jnp.sum(tw, axis=-1, keepdims=True), tw)

    flat_ids = ti.reshape(-1).astype(jnp.int32)
    oh = flat_ids[:, None] == jnp.arange(_E, dtype=jnp.int32)[None, :]
    ohi = oh.astype(jnp.int32)
    counts = ohi.sum(axis=0)
    rank = jnp.where(oh, jnp.cumsum(ohi, axis=0) - 1, 0).sum(axis=1)
    nblk = (counts + _BM - 1) // _BM
    bstart = jnp.concatenate(
        [jnp.zeros((1,), jnp.int32), jnp.cumsum(nblk)[:-1].astype(jnp.int32)])
    dest = bstart[flat_ids] * _BM + rank
    token = jnp.arange(_M, dtype=jnp.int32) // _K
    sorted_token = (jnp.arange(_NBM, dtype=jnp.int32) % _T).at[dest].set(token)
    bend = jnp.cumsum(nblk)
    block_expert = jnp.sum(
        (jnp.arange(_NB, dtype=jnp.int32)[:, None] >= bend[None, :])
        .astype(jnp.int32), axis=1)
    block_expert = jnp.minimum(block_expert, _E - 1)

    s13k = jnp.repeat(w13_weight_scale_inv, _BN, axis=2)
    s2k = jnp.repeat(w2_weight_scale_inv, _BN, axis=2)
    w13f, w2f = _dequant_all(w13_weight, s13k, w2_weight, s2k)

    xs = _sc_gather_rows(x, sorted_token)

    o_sorted = _ffn(block_expert, xs, w13f, w2f)
    d = dest.reshape(_T, _K)
    oa, ob = _sc_combine(o_sorted, d[:, 0], d[:, 1])
    return _tc_scale_add(oa, ob, tw[:, :1], tw[:, 1:])
